# baseline (speedup 1.0000x reference)
import functools

import jax
import jax.numpy as jnp
from jax import lax
from jax.experimental import pallas as pl
from jax.experimental.pallas import tpu as pltpu
from jax.experimental.pallas import tpu_sc as plsc

D = 32
B = 16384
V = 1000000

_info = plsc.get_sparse_core_info()
_NC, _NS = _info.num_cores, _info.num_subcores
NW = _NC * _NS
BPW = B // NW
GS = 16
NS = 20
NBLK = BPW // 128

_mesh = plsc.VectorSubcoreMesh(core_axis_name="c", subcore_axis_name="s")


@functools.partial(
    pl.kernel,
    mesh=_mesh,
    out_type=jax.ShapeDtypeStruct((D, B), jnp.float32),
    scratch_types=[
        pltpu.VMEM((BPW,), jnp.int32),
        pltpu.VMEM((NS, D, 128), jnp.float32),
        pltpu.VMEM((D, 128), jnp.float32),
        pltpu.SemaphoreType.DMA,
        pltpu.SemaphoreType.DMA,
    ],
    compiler_params=pltpu.CompilerParams(needs_layout_passes=False),
)
def _gather_kernel(idx_hbm, embT_hbm, outT_hbm, idx_v, blk_v, ob_v, gsem, osem):
    wid = lax.axis_index("s") * _NC + lax.axis_index("c")
    base = wid * BPW
    pltpu.sync_copy(idx_hbm.at[pl.ds(base, BPW)], idx_v)
    iota = lax.iota(jnp.int32, 16)

    def block(blki, carry):
        bb = blki * 128
        lanes = [None] * 128
        w0s = [None] * 128
        copies = [None] * 128

        def load_burst(sub):
            rv = idx_v[pl.ds(bb + sub * GS, GS)]
            for i in range(GS):
                e = sub * GS + i
                r = rv[i]
                w0 = pl.multiple_of(
                    lax.shift_left(lax.shift_right_logical(r, 7), 7), 128
                )
                w0s[e] = w0
                lanes[e] = r - w0

        def issue(e):
            copies[e] = pltpu.async_copy(
                embT_hbm.at[:, pl.ds(w0s[e], 128)], blk_v.at[e % NS], gsem
            )

        load_burst(0)
        load_burst(1)
        for e in range(NS):
            issue(e)
        for e in range(128):
            if e % GS == 0 and e // GS + 2 < 128 // GS:
                load_burst(e // GS + 2)
            copies[e].wait()
            lane = jnp.full((16,), lanes[e], jnp.int32)
            row = jnp.full((16,), e % NS, jnp.int32)
            col = jnp.full((16,), e, jnp.int32)
            lo = plsc.load_gather(blk_v, [row, iota, lane])
            hi = plsc.load_gather(blk_v, [row, iota + 16, lane])
            plsc.store_scatter(ob_v, [iota, col], lo)
            plsc.store_scatter(ob_v, [iota + 16, col], hi)
            if e + NS < 128:
                issue(e + NS)
        pltpu.async_copy(
            ob_v, outT_hbm.at[:, pl.ds(base + bb, 128)], osem
        ).wait()
        return carry

    lax.fori_loop(0, NBLK, block, 0)


def kernel(node_idx, emb):
    outT = _gather_kernel(node_idx.astype(jnp.int32), emb.T)
    return outT.T


# speedup vs baseline: 1.0129x; 1.0129x over previous
import functools

import jax
import jax.numpy as jnp
from jax import lax
from jax.experimental import pallas as pl
from jax.experimental.pallas import tpu as pltpu
from jax.experimental.pallas import tpu_sc as plsc

D = 32
B = 16384
V = 1000000

_info = plsc.get_sparse_core_info()
_NC, _NS = _info.num_cores, _info.num_subcores
NW = _NC * _NS
BPW = B // NW
GS = 16
NS = 16
NBLK = BPW // 128

_mesh = plsc.VectorSubcoreMesh(core_axis_name="c", subcore_axis_name="s")


@functools.partial(
    pl.kernel,
    mesh=_mesh,
    out_type=jax.ShapeDtypeStruct((D, B), jnp.float32),
    scratch_types=[
        pltpu.VMEM((BPW,), jnp.int32),
        pltpu.VMEM((NS, D, 128), jnp.float32),
        pltpu.VMEM((D, 128), jnp.float32),
        pltpu.SemaphoreType.DMA,
        pltpu.SemaphoreType.DMA,
    ],
    compiler_params=pltpu.CompilerParams(needs_layout_passes=False),
)
def _gather_kernel(idx_hbm, embT_hbm, outT_hbm, idx_v, blk_v, ob_v, gsem, osem):
    wid = lax.axis_index("s") * _NC + lax.axis_index("c")
    base = wid * BPW
    pltpu.sync_copy(idx_hbm.at[pl.ds(base, BPW)], idx_v)
    iota = lax.iota(jnp.int32, 16)

    def block(blki, carry):
        bb = blki * 128
        lanes = [None] * 128
        w0s = [None] * 128
        copies = [None] * 128

        def load_burst(sub):
            rv = idx_v[pl.ds(bb + sub * GS, GS)]
            for i in range(GS):
                e = sub * GS + i
                r = rv[i]
                w0 = pl.multiple_of(
                    lax.shift_left(lax.shift_right_logical(r, 7), 7), 128
                )
                w0s[e] = w0
                lanes[e] = r - w0

        def issue(e):
            copies[e] = pltpu.async_copy(
                embT_hbm.at[:, pl.ds(w0s[e], 128)], blk_v.at[e % NS], gsem
            )

        load_burst(0)
        load_burst(1)
        for e in range(NS):
            issue(e)
        for e in range(128):
            if e % GS == 0 and e // GS + 2 < 128 // GS:
                load_burst(e // GS + 2)
            copies[e].wait()
            lane = jnp.full((16,), lanes[e], jnp.int32)
            row = jnp.full((16,), e % NS, jnp.int32)
            col = jnp.full((16,), e, jnp.int32)
            lo = plsc.load_gather(blk_v, [row, iota, lane])
            hi = plsc.load_gather(blk_v, [row, iota + 16, lane])
            plsc.store_scatter(ob_v, [iota, col], lo)
            plsc.store_scatter(ob_v, [iota + 16, col], hi)
            if e + NS < 128:
                issue(e + NS)
        pltpu.async_copy(
            ob_v, outT_hbm.at[:, pl.ds(base + bb, 128)], osem
        ).wait()
        return carry

    lax.fori_loop(0, NBLK, block, 0)


def kernel(node_idx, emb):
    outT = _gather_kernel(node_idx.astype(jnp.int32), emb.T)
    return outT.T
